# baseline (speedup 1.0000x reference)
import math

import jax
import jax.numpy as jnp
from jax.experimental import pallas as pl
from jax.experimental.pallas import tpu as pltpu

B, C, H, W = 512, 64, 8, 8
S = H * W
E, NH, MEM = 64, 8, 8
HD = E // NH
THD = H * W * HD
AMB = 5.0
LN_EPS = 1e-5

BB = 32
GRID = B // BB
NEG = -1e30


def _im2col(xt):
    cin = xt.shape[-1]
    s_idx = jax.lax.broadcasted_iota(jnp.int32, (1, S, 1), 1)
    yy = s_idx // W
    xx = s_idx % W
    cols = []
    for ky in range(3):
        for kx in range(3):
            dy, dx = ky - 1, kx - 1
            off = dy * W + dx
            shifted = jnp.roll(xt, -off, axis=1) if off else xt
            valid = ((yy + dy >= 0) & (yy + dy < H)
                     & (xx + dx >= 0) & (xx + dx < W))
            cols.append(jnp.where(valid, shifted, 0.0))
    del cin
    return jnp.concatenate(cols, axis=-1)


def _merge_thd(x):
    x5 = x.reshape(BB, NH, HD, S)
    return jnp.concatenate([x5[:, :, hd, :] for hd in range(HD)], axis=-1)


def _split_thd(x):
    parts = [x[:, :, hd * S:(hd + 1) * S] for hd in range(HD)]
    return jnp.stack(parts, axis=2).reshape(BB, E, S)


def _cell_kernel(xh_ref, c_ref, k_ref, v_ref, mask_ref,
                 wgk_ref, w3_ref, b1_ref, b2_ref, b3_ref,
                 posw_ref, posb_ref, lnw_ref, lnb_ref,
                 hc_out, k_out, v_out):
    f32 = jnp.float32
    comb = xh_ref[...]
    xt = comb[:, :, 0:C]
    mc = _im2col(comb).reshape(BB * S, 9 * (C + E)).astype(jnp.bfloat16)
    gk = jnp.dot(mc, wgk_ref[...], preferred_element_type=f32)
    gates = gk[:, 0:5 * E] + b1_ref[...]
    kqv = gk[:, 5 * E:8 * E] + b2_ref[...]

    gi = jax.nn.sigmoid(gates[:, 0:E])
    gf = jax.nn.sigmoid(gates[:, E:2 * E])
    go = jax.nn.sigmoid(gates[:, 2 * E:3 * E])
    gg = jnp.tanh(gates[:, 3 * E:4 * E])
    ga = jax.nn.sigmoid(gates[:, 4 * E:5 * E])
    ct = c_ref[...].reshape(BB * S, E)
    c1 = gf * ct + gi * gg

    def to_cf(sl):
        return jnp.swapaxes(sl.reshape(BB, S, E), 1, 2)

    k_new = _merge_thd(to_cf(kqv[:, 0:E]))
    q_thd = _merge_thd(to_cf(kqv[:, E:2 * E])) * (1.0 / math.sqrt(THD))
    v_new = _merge_thd(to_cf(kqv[:, 2 * E:3 * E]))

    posw = posw_ref[...]
    kfull = jnp.concatenate([k_ref[:, 1:MEM], k_new[:, None]], axis=1) \
        + posw[None]
    vfull = jnp.concatenate([v_ref[:, 1:MEM], v_new[:, None]], axis=1)
    k_out[...] = kfull
    v_out[...] = vfull

    qks = []
    for m in range(MEM):
        qks.append(jnp.sum(kfull[:, m] * q_thd, axis=2))
    qk = jnp.stack(qks, axis=1)
    m_row = jax.lax.broadcasted_iota(jnp.int32, (1, MEM, 1), 1)
    additive = jnp.where(mask_ref[...] > 0.0, NEG, 0.0)
    additive = jnp.where(m_row == MEM - 1, AMB, additive)
    scores = qk + additive + posb_ref[...][None]
    mx = jnp.max(scores, axis=1, keepdims=True)
    ex = jnp.exp(scores - mx)
    wgt = ex / jnp.sum(ex, axis=1, keepdims=True)

    attn = wgt[:, 0, :, None] * vfull[:, 0]
    for m in range(1, MEM):
        attn = attn + wgt[:, m, :, None] * vfull[:, m]
    at_sl = jnp.swapaxes(_split_thd(attn), 1, 2)

    m3 = _im2col(at_sl).reshape(BB * S, 9 * E).astype(jnp.bfloat16)
    out = (jnp.dot(m3, w3_ref[...], preferred_element_type=f32)
           + b3_ref[...] + xt.reshape(BB * S, C))
    out3 = out.reshape(BB, S, E)
    mu = jnp.sum(out3, axis=(1, 2), keepdims=True) * (1.0 / (S * E))
    dev = out3 - mu
    var = jnp.sum(dev * dev, axis=(1, 2), keepdims=True) * (1.0 / (S * E))
    norm = dev * jax.lax.rsqrt(var + LN_EPS) * lnw_ref[...][None] \
        + lnb_ref[...][None]
    norm = norm.reshape(BB * S, E)

    c2 = c1 + ga * jnp.tanh(norm)
    hn = go * jnp.tanh(c2)
    hc_out[...] = jnp.concatenate(
        [hn.reshape(BB, S, E), c2.reshape(BB, S, E)], axis=-1)


def kernel(input, h_cur, c_cur, concat_k, concat_v, attn_mask, conv_w, conv_b,
           proj_w, proj_b, out_w, out_b, ln_w, ln_b, pos_w, pos_b,
           interpret=False):
    xh = jnp.concatenate(
        [input.reshape(B, C, S), h_cur.reshape(B, E, S)],
        axis=1).transpose(0, 2, 1)
    c = c_cur.reshape(B, E, S).transpose(0, 2, 1)
    maskf = attn_mask.reshape(B, NH, MEM).transpose(0, 2, 1).astype(jnp.float32)
    w1 = conv_w.transpose(2, 3, 1, 0).reshape(9 * (C + E), 5 * E)
    w2 = proj_w.transpose(2, 3, 1, 0).reshape(9, C, 3 * E)
    w2p = jnp.concatenate(
        [w2, jnp.zeros((9, E, 3 * E), w2.dtype)], axis=1).reshape(
            9 * (C + E), 3 * E)
    wgk = jnp.concatenate([w1, w2p], axis=1).astype(jnp.bfloat16)
    w3 = out_w.transpose(2, 3, 1, 0).reshape(9 * E, E).astype(jnp.bfloat16)
    b1 = conv_b.reshape(1, 5 * E)
    b2 = proj_b.reshape(1, 3 * E)
    b3 = out_b.reshape(1, E)
    posw = pos_w.reshape(MEM, NH, THD)
    lnw = ln_w.reshape(E, S).T
    lnb = ln_b.reshape(E, S).T

    blk = lambda shp: pl.BlockSpec(shp, lambda i: (i,) + (0,) * (len(shp) - 1))
    full = lambda arr: pl.BlockSpec(arr.shape, lambda i: (0,) * arr.ndim)

    hc, k_o, v_o = pl.pallas_call(
        _cell_kernel,
        grid=(GRID,),
        in_specs=[
            blk((BB, S, C + E)), blk((BB, S, E)),
            blk((BB, MEM, NH, THD)), blk((BB, MEM, NH, THD)),
            blk((BB, MEM, NH)),
            full(wgk), full(w3),
            full(b1), full(b2), full(b3),
            full(posw), full(pos_b), full(lnw), full(lnb),
        ],
        out_specs=[
            blk((BB, S, 2 * E)),
            blk((BB, MEM, NH, THD)), blk((BB, MEM, NH, THD)),
        ],
        out_shape=[
            jax.ShapeDtypeStruct((B, S, 2 * E), jnp.float32),
            jax.ShapeDtypeStruct((B, MEM, NH, THD), jnp.float32),
            jax.ShapeDtypeStruct((B, MEM, NH, THD), jnp.float32),
        ],
        compiler_params=pltpu.CompilerParams(
            dimension_semantics=("arbitrary",),
            vmem_limit_bytes=56 * 1024 * 1024,
        ),
        name="conv_attn_lstm_cell",
        interpret=interpret,
    )(xh, c, concat_k, concat_v, maskf, wgk, w3, b1, b2, b3,
      posw, pos_b, lnw, lnb)
    hc_t = hc.transpose(0, 2, 1)
    return (hc_t[:, 0:E].reshape(B, E, H, W),
            hc_t[:, E:2 * E].reshape(B, E, H, W), k_o, v_o)


# speedup vs baseline: 1.1051x; 1.1051x over previous
import math

import jax
import jax.numpy as jnp
from jax.experimental import pallas as pl
from jax.experimental.pallas import tpu as pltpu

B, C, H, W = 512, 64, 8, 8
S = H * W
E, NH, MEM = 64, 8, 8
HD = E // NH
THD = H * W * HD
AMB = 5.0
LN_EPS = 1e-5

BB = 32
GRID = B // BB
NEG = -1e30


def _im2col(xt):
    cin = xt.shape[-1]
    s_idx = jax.lax.broadcasted_iota(jnp.int32, (1, S, 1), 1)
    yy = s_idx // W
    xx = s_idx % W
    cols = []
    for ky in range(3):
        for kx in range(3):
            dy, dx = ky - 1, kx - 1
            off = dy * W + dx
            shifted = jnp.roll(xt, -off, axis=1) if off else xt
            valid = ((yy + dy >= 0) & (yy + dy < H)
                     & (xx + dx >= 0) & (xx + dx < W))
            cols.append(jnp.where(valid, shifted, 0.0))
    del cin
    return jnp.concatenate(cols, axis=-1)


def _merge_thd(x):
    x5 = x.reshape(BB, NH, HD, S)
    return jnp.concatenate([x5[:, :, hd, :] for hd in range(HD)], axis=-1)


def _split_thd(x):
    parts = [x[:, :, hd * S:(hd + 1) * S] for hd in range(HD)]
    return jnp.stack(parts, axis=2).reshape(BB, E, S)


def _cell_kernel(x_ref, h_ref, c_ref, k_ref, v_ref, mask_ref,
                 wgk_ref, w3_ref, b1_ref, b2_ref, b3_ref,
                 posw_ref, posb_ref, lnw_ref, lnb_ref,
                 h_out, c_out, k_out, v_out):
    f32 = jnp.float32
    xt = x_ref[...]
    comb = jnp.concatenate([xt, h_ref[...]], axis=-1)
    mc = _im2col(comb).reshape(BB * S, 9 * (C + E)).astype(jnp.bfloat16)
    gk = jnp.dot(mc, wgk_ref[...], preferred_element_type=f32)
    gates = gk[:, 0:5 * E] + b1_ref[...]
    kqv = gk[:, 5 * E:8 * E] + b2_ref[...]

    gi = jax.nn.sigmoid(gates[:, 0:E])
    gf = jax.nn.sigmoid(gates[:, E:2 * E])
    go = jax.nn.sigmoid(gates[:, 2 * E:3 * E])
    gg = jnp.tanh(gates[:, 3 * E:4 * E])
    ga = jax.nn.sigmoid(gates[:, 4 * E:5 * E])
    ct = c_ref[...].reshape(BB * S, E)
    c1 = gf * ct + gi * gg

    kqv_t = jnp.swapaxes(kqv.reshape(BB, S, 3 * E), 1, 2)
    k_new = _merge_thd(kqv_t[:, 0:E])
    q_thd = _merge_thd(kqv_t[:, E:2 * E]) * (1.0 / math.sqrt(THD))
    v_new = _merge_thd(kqv_t[:, 2 * E:3 * E])

    posw = posw_ref[...]
    qks = []
    for m in range(MEM):
        km = (k_ref[:, m + 1] if m < MEM - 1 else k_new) + posw[m][None]
        k_out[:, m] = km
        qks.append(jnp.sum(km * q_thd, axis=2))
        v_out[:, m] = v_ref[:, m + 1] if m < MEM - 1 else v_new
    qk = jnp.stack(qks, axis=1)
    m_row = jax.lax.broadcasted_iota(jnp.int32, (1, MEM, 1), 1)
    additive = jnp.where(mask_ref[...] > 0.0, NEG, 0.0)
    additive = jnp.where(m_row == MEM - 1, AMB, additive)
    scores = qk + additive + posb_ref[...][None]
    mx = jnp.max(scores, axis=1, keepdims=True)
    ex = jnp.exp(scores - mx)
    wgt = ex / jnp.sum(ex, axis=1, keepdims=True)

    attn = wgt[:, MEM - 1, :, None] * v_new
    for m in range(MEM - 1):
        attn = attn + wgt[:, m, :, None] * v_ref[:, m + 1]
    at_sl = jnp.swapaxes(_split_thd(attn), 1, 2)

    m3 = _im2col(at_sl).reshape(BB * S, 9 * E).astype(jnp.bfloat16)
    out = (jnp.dot(m3, w3_ref[...], preferred_element_type=f32)
           + b3_ref[...] + xt.reshape(BB * S, C))
    out3 = out.reshape(BB, S, E)
    mu = jnp.sum(out3, axis=(1, 2), keepdims=True) * (1.0 / (S * E))
    dev = out3 - mu
    var = jnp.sum(dev * dev, axis=(1, 2), keepdims=True) * (1.0 / (S * E))
    norm = dev * jax.lax.rsqrt(var + LN_EPS) * lnw_ref[...][None] \
        + lnb_ref[...][None]
    norm = norm.reshape(BB * S, E)

    c2 = c1 + ga * jnp.tanh(norm)
    hn = go * jnp.tanh(c2)
    h_out[...] = hn.reshape(BB, S, E)
    c_out[...] = c2.reshape(BB, S, E)


def kernel(input, h_cur, c_cur, concat_k, concat_v, attn_mask, conv_w, conv_b,
           proj_w, proj_b, out_w, out_b, ln_w, ln_b, pos_w, pos_b,
           interpret=False):
    x = input.reshape(B, C, S).transpose(0, 2, 1)
    h = h_cur.reshape(B, E, S).transpose(0, 2, 1)
    c = c_cur.reshape(B, E, S).transpose(0, 2, 1)
    maskf = attn_mask.reshape(B, NH, MEM).transpose(0, 2, 1).astype(jnp.float32)
    w1 = conv_w.transpose(2, 3, 1, 0).reshape(9 * (C + E), 5 * E)
    w2 = proj_w.transpose(2, 3, 1, 0).reshape(9, C, 3 * E)
    w2p = jnp.concatenate(
        [w2, jnp.zeros((9, E, 3 * E), w2.dtype)], axis=1).reshape(
            9 * (C + E), 3 * E)
    wgk = jnp.concatenate([w1, w2p], axis=1).astype(jnp.bfloat16)
    w3 = out_w.transpose(2, 3, 1, 0).reshape(9 * E, E).astype(jnp.bfloat16)
    b1 = conv_b.reshape(1, 5 * E)
    b2 = proj_b.reshape(1, 3 * E)
    b3 = out_b.reshape(1, E)
    posw = pos_w.reshape(MEM, NH, THD)
    lnw = ln_w.reshape(E, S).T
    lnb = ln_b.reshape(E, S).T

    blk = lambda shp: pl.BlockSpec(shp, lambda i: (i,) + (0,) * (len(shp) - 1))
    full = lambda arr: pl.BlockSpec(arr.shape, lambda i: (0,) * arr.ndim)

    h_n, c_n, k_o, v_o = pl.pallas_call(
        _cell_kernel,
        grid=(GRID,),
        in_specs=[
            blk((BB, S, C)), blk((BB, S, E)), blk((BB, S, E)),
            blk((BB, MEM, NH, THD)), blk((BB, MEM, NH, THD)),
            blk((BB, MEM, NH)),
            full(wgk), full(w3),
            full(b1), full(b2), full(b3),
            full(posw), full(pos_b), full(lnw), full(lnb),
        ],
        out_specs=[
            blk((BB, S, E)), blk((BB, S, E)),
            blk((BB, MEM, NH, THD)), blk((BB, MEM, NH, THD)),
        ],
        out_shape=[
            jax.ShapeDtypeStruct((B, S, E), jnp.float32),
            jax.ShapeDtypeStruct((B, S, E), jnp.float32),
            jax.ShapeDtypeStruct((B, MEM, NH, THD), jnp.float32),
            jax.ShapeDtypeStruct((B, MEM, NH, THD), jnp.float32),
        ],
        compiler_params=pltpu.CompilerParams(
            dimension_semantics=("arbitrary",),
            vmem_limit_bytes=56 * 1024 * 1024,
        ),
        name="conv_attn_lstm_cell",
        interpret=interpret,
    )(x, h, c, concat_k, concat_v, maskf, wgk, w3, b1, b2, b3,
      posw, pos_b, lnw, lnb)
    return (h_n.transpose(0, 2, 1).reshape(B, E, H, W),
            c_n.transpose(0, 2, 1).reshape(B, E, H, W), k_o, v_o)


# speedup vs baseline: 1.1399x; 1.0315x over previous
import math

import jax
import jax.numpy as jnp
from jax.experimental import pallas as pl
from jax.experimental.pallas import tpu as pltpu

B, C, H, W = 512, 64, 8, 8
S = H * W
E, NH, MEM = 64, 8, 8
HD = E // NH
THD = H * W * HD
AMB = 5.0
LN_EPS = 1e-5

BB = 32
GRID = B // BB
NEG = -1e30


def _im2col(xt):
    cin = xt.shape[-1]
    s_idx = jax.lax.broadcasted_iota(jnp.int32, (1, S, 1), 1)
    yy = s_idx // W
    xx = s_idx % W
    cols = []
    for ky in range(3):
        for kx in range(3):
            dy, dx = ky - 1, kx - 1
            off = dy * W + dx
            shifted = jnp.roll(xt, -off, axis=1) if off else xt
            valid = ((yy + dy >= 0) & (yy + dy < H)
                     & (xx + dx >= 0) & (xx + dx < W))
            cols.append(jnp.where(valid, shifted, 0.0))
    del cin
    return jnp.concatenate(cols, axis=-1)


def _merge_thd(x):
    x5 = x.reshape(BB, NH, HD, S)
    return jnp.concatenate([x5[:, :, hd, :] for hd in range(HD)], axis=-1)


def _split_thd(x):
    parts = [x[:, :, hd * S:(hd + 1) * S] for hd in range(HD)]
    return jnp.stack(parts, axis=2).reshape(BB, E, S)


def _cell_kernel(x_ref, h_ref, c_ref, k_ref, v_ref, mask_ref,
                 wgk_ref, w3_ref, b1_ref, b2_ref, b3_ref,
                 posw_ref, posb_ref, lnw_ref, lnb_ref,
                 h_out, c_out, k_out, v_out):
    f32 = jnp.float32
    xt = x_ref[...]
    comb = jnp.concatenate([xt, h_ref[...]], axis=-1)
    mc = _im2col(comb).reshape(BB * S, 9 * (C + E)).astype(jnp.bfloat16)
    gk = jnp.dot(mc, wgk_ref[...], preferred_element_type=f32)
    gates = gk[:, 0:5 * E] + b1_ref[...]
    kqv = gk[:, 5 * E:8 * E] + b2_ref[...]

    gi = jax.nn.sigmoid(gates[:, 0:E])
    gf = jax.nn.sigmoid(gates[:, E:2 * E])
    go = jax.nn.sigmoid(gates[:, 2 * E:3 * E])
    gg = jnp.tanh(gates[:, 3 * E:4 * E])
    ga = jax.nn.sigmoid(gates[:, 4 * E:5 * E])
    ct = c_ref[...].reshape(BB * S, E)
    c1 = gf * ct + gi * gg

    def to_cf(sl):
        return jnp.swapaxes(sl.reshape(BB, S, E), 1, 2)

    k_new = _merge_thd(to_cf(kqv[:, 0:E]))
    q_thd = _merge_thd(to_cf(kqv[:, E:2 * E])) * (1.0 / math.sqrt(THD))
    v_new = _merge_thd(to_cf(kqv[:, 2 * E:3 * E]))

    posw = posw_ref[...]
    kfull = jnp.concatenate([k_ref[:, 1:MEM], k_new[:, None]], axis=1) \
        + posw[None]
    vfull = jnp.concatenate([v_ref[:, 1:MEM], v_new[:, None]], axis=1)
    k_out[...] = kfull
    v_out[...] = vfull

    qks = []
    for m in range(MEM):
        qks.append(jnp.sum(kfull[:, m] * q_thd, axis=2))
    qk = jnp.stack(qks, axis=1)
    m_row = jax.lax.broadcasted_iota(jnp.int32, (1, MEM, 1), 1)
    additive = jnp.where(mask_ref[...] > 0.0, NEG, 0.0)
    additive = jnp.where(m_row == MEM - 1, AMB, additive)
    scores = qk + additive + posb_ref[...][None]
    mx = jnp.max(scores, axis=1, keepdims=True)
    ex = jnp.exp(scores - mx)
    wgt = ex / jnp.sum(ex, axis=1, keepdims=True)

    attn = wgt[:, 0, :, None] * vfull[:, 0]
    for m in range(1, MEM):
        attn = attn + wgt[:, m, :, None] * vfull[:, m]
    at_sl = jnp.swapaxes(_split_thd(attn), 1, 2)

    m3 = _im2col(at_sl).reshape(BB * S, 9 * E).astype(jnp.bfloat16)
    out = (jnp.dot(m3, w3_ref[...], preferred_element_type=f32)
           + b3_ref[...] + xt.reshape(BB * S, C))
    out3 = out.reshape(BB, S, E)
    mu = jnp.sum(out3, axis=(1, 2), keepdims=True) * (1.0 / (S * E))
    dev = out3 - mu
    var = jnp.sum(dev * dev, axis=(1, 2), keepdims=True) * (1.0 / (S * E))
    norm = dev * jax.lax.rsqrt(var + LN_EPS) * lnw_ref[...][None] \
        + lnb_ref[...][None]
    norm = norm.reshape(BB * S, E)

    c2 = c1 + ga * jnp.tanh(norm)
    hn = go * jnp.tanh(c2)
    h_out[...] = hn.reshape(BB, S, E)
    c_out[...] = c2.reshape(BB, S, E)


def kernel(input, h_cur, c_cur, concat_k, concat_v, attn_mask, conv_w, conv_b,
           proj_w, proj_b, out_w, out_b, ln_w, ln_b, pos_w, pos_b):
    x = input.reshape(B, C, S).transpose(0, 2, 1)
    h = h_cur.reshape(B, E, S).transpose(0, 2, 1)
    c = c_cur.reshape(B, E, S).transpose(0, 2, 1)
    maskf = attn_mask.reshape(B, NH, MEM).transpose(0, 2, 1).astype(jnp.float32)
    w1 = conv_w.transpose(2, 3, 1, 0).reshape(9 * (C + E), 5 * E)
    w2 = proj_w.transpose(2, 3, 1, 0).reshape(9, C, 3 * E)
    w2p = jnp.concatenate(
        [w2, jnp.zeros((9, E, 3 * E), w2.dtype)], axis=1).reshape(
            9 * (C + E), 3 * E)
    wgk = jnp.concatenate([w1, w2p], axis=1).astype(jnp.bfloat16)
    w3 = out_w.transpose(2, 3, 1, 0).reshape(9 * E, E).astype(jnp.bfloat16)
    b1 = conv_b.reshape(1, 5 * E)
    b2 = proj_b.reshape(1, 3 * E)
    b3 = out_b.reshape(1, E)
    posw = pos_w.reshape(MEM, NH, THD)
    lnw = ln_w.reshape(E, S).T
    lnb = ln_b.reshape(E, S).T

    blk = lambda shp: pl.BlockSpec(shp, lambda i: (i,) + (0,) * (len(shp) - 1))
    full = lambda arr: pl.BlockSpec(arr.shape, lambda i: (0,) * arr.ndim)

    h_n, c_n, k_o, v_o = pl.pallas_call(
        _cell_kernel,
        grid=(GRID,),
        in_specs=[
            blk((BB, S, C)), blk((BB, S, E)), blk((BB, S, E)),
            blk((BB, MEM, NH, THD)), blk((BB, MEM, NH, THD)),
            blk((BB, MEM, NH)),
            full(wgk), full(w3),
            full(b1), full(b2), full(b3),
            full(posw), full(pos_b), full(lnw), full(lnb),
        ],
        out_specs=[
            blk((BB, S, E)), blk((BB, S, E)),
            blk((BB, MEM, NH, THD)), blk((BB, MEM, NH, THD)),
        ],
        out_shape=[
            jax.ShapeDtypeStruct((B, S, E), jnp.float32),
            jax.ShapeDtypeStruct((B, S, E), jnp.float32),
            jax.ShapeDtypeStruct((B, MEM, NH, THD), jnp.float32),
            jax.ShapeDtypeStruct((B, MEM, NH, THD), jnp.float32),
        ],
        compiler_params=pltpu.CompilerParams(
            dimension_semantics=("arbitrary",),
            vmem_limit_bytes=56 * 1024 * 1024,
        ),
        name="conv_attn_lstm_cell",
    )(x, h, c, concat_k, concat_v, maskf, wgk, w3, b1, b2, b3,
      posw, pos_b, lnw, lnb)
    return (h_n.transpose(0, 2, 1).reshape(B, E, H, W),
            c_n.transpose(0, 2, 1).reshape(B, E, H, W), k_o, v_o)


# speedup vs baseline: 1.1407x; 1.0007x over previous
import math

import jax
import jax.numpy as jnp
from jax.experimental import pallas as pl
from jax.experimental.pallas import tpu as pltpu

B, C, H, W = 512, 64, 8, 8
S = H * W
E, NH, MEM = 64, 8, 8
HD = E // NH
THD = H * W * HD
AMB = 5.0
LN_EPS = 1e-5

BB = 32
GRID = B // BB
NEG = -1e30


def _im2col(xt):
    s_idx = jax.lax.broadcasted_iota(jnp.int32, (1, S, 1), 1)
    yy = s_idx // W
    xx = s_idx % W
    cols = []
    for ky in range(3):
        for kx in range(3):
            dy, dx = ky - 1, kx - 1
            off = dy * W + dx
            shifted = jnp.roll(xt, -off, axis=1) if off else xt
            valid = ((yy + dy >= 0) & (yy + dy < H)
                     & (xx + dx >= 0) & (xx + dx < W))
            cols.append(jnp.where(valid, shifted, 0.0))
    return jnp.concatenate(cols, axis=-1)


def _merge_thd(x):
    x5 = x.reshape(BB, NH, HD, S)
    return jnp.concatenate([x5[:, :, hd, :] for hd in range(HD)], axis=-1)


def _split_thd(x):
    parts = [x[:, :, hd * S:(hd + 1) * S] for hd in range(HD)]
    return jnp.stack(parts, axis=2).reshape(BB, E, S)


def _cell_kernel(x_ref, h_ref, c_ref, k_ref, v_ref, mask_ref,
                 wgk_ref, w3_ref, b1_ref, b2_ref, b3_ref,
                 posw_ref, posb_ref, lnw_ref, lnb_ref,
                 h_out, c_out, k_out, v_out):
    f32 = jnp.float32
    xt = x_ref[...]
    comb = jnp.concatenate([xt, h_ref[...]], axis=-1)
    mc = _im2col(comb).reshape(BB * S, 9 * (C + E)).astype(jnp.bfloat16)
    gk = jnp.dot(mc, wgk_ref[...], preferred_element_type=f32)
    gates = gk[:, 0:5 * E] + b1_ref[...]
    kqv = gk[:, 5 * E:8 * E] + b2_ref[...]

    gi = jax.nn.sigmoid(gates[:, 0:E])
    gf = jax.nn.sigmoid(gates[:, E:2 * E])
    go = jax.nn.sigmoid(gates[:, 2 * E:3 * E])
    gg = jnp.tanh(gates[:, 3 * E:4 * E])
    ga = jax.nn.sigmoid(gates[:, 4 * E:5 * E])
    ct = c_ref[...].reshape(BB * S, E)
    c1 = gf * ct + gi * gg

    def to_cf(sl):
        return jnp.swapaxes(sl.reshape(BB, S, E), 1, 2)

    k_new = _merge_thd(to_cf(kqv[:, 0:E]))
    q_thd = _merge_thd(to_cf(kqv[:, E:2 * E])) * (1.0 / math.sqrt(THD))
    v_new = _merge_thd(to_cf(kqv[:, 2 * E:3 * E]))

    posw = posw_ref[...]
    kfull = jnp.concatenate([k_ref[:, 1:MEM], k_new[:, None]], axis=1) \
        + posw[None]
    vfull = jnp.concatenate([v_ref[:, 1:MEM], v_new[:, None]], axis=1)
    k_out[...] = kfull
    v_out[...] = vfull

    qks = []
    for m in range(MEM):
        qks.append(jnp.sum(kfull[:, m] * q_thd, axis=2))
    qk = jnp.stack(qks, axis=1)
    m_row = jax.lax.broadcasted_iota(jnp.int32, (1, MEM, 1), 1)
    additive = jnp.where(mask_ref[...] > 0.0, NEG, 0.0)
    additive = jnp.where(m_row == MEM - 1, AMB, additive)
    scores = qk + additive + posb_ref[...][None]
    mx = jnp.max(scores, axis=1, keepdims=True)
    ex = jnp.exp(scores - mx)
    wgt = ex / jnp.sum(ex, axis=1, keepdims=True)

    attn = wgt[:, 0, :, None] * vfull[:, 0]
    for m in range(1, MEM):
        attn = attn + wgt[:, m, :, None] * vfull[:, m]
    at_sl = jnp.swapaxes(_split_thd(attn), 1, 2)

    m3 = _im2col(at_sl).reshape(BB * S, 9 * E).astype(jnp.bfloat16)
    out = (jnp.dot(m3, w3_ref[...], preferred_element_type=f32)
           + b3_ref[...] + xt.reshape(BB * S, C))
    out3 = out.reshape(BB, S, E)
    mu = jnp.sum(out3, axis=(1, 2), keepdims=True) * (1.0 / (S * E))
    dev = out3 - mu
    var = jnp.sum(dev * dev, axis=(1, 2), keepdims=True) * (1.0 / (S * E))
    norm = dev * jax.lax.rsqrt(var + LN_EPS) * lnw_ref[...][None] \
        + lnb_ref[...][None]
    norm = norm.reshape(BB * S, E)

    c2 = c1 + ga * jnp.tanh(norm)
    hn = go * jnp.tanh(c2)
    h_out[...] = hn.reshape(BB, S, E)
    c_out[...] = c2.reshape(BB, S, E)


def kernel(input, h_cur, c_cur, concat_k, concat_v, attn_mask, conv_w, conv_b,
           proj_w, proj_b, out_w, out_b, ln_w, ln_b, pos_w, pos_b):
    x = input.reshape(B, C, S).transpose(0, 2, 1)
    h = h_cur.reshape(B, E, S).transpose(0, 2, 1)
    c = c_cur.reshape(B, E, S).transpose(0, 2, 1)
    maskf = attn_mask.reshape(B, NH, MEM).transpose(0, 2, 1).astype(jnp.float32)
    w1 = conv_w.transpose(2, 3, 1, 0).reshape(9 * (C + E), 5 * E)
    w2 = proj_w.transpose(2, 3, 1, 0).reshape(9, C, 3 * E)
    w2p = jnp.concatenate(
        [w2, jnp.zeros((9, E, 3 * E), w2.dtype)], axis=1).reshape(
            9 * (C + E), 3 * E)
    wgk = jnp.concatenate([w1, w2p], axis=1).astype(jnp.bfloat16)
    w3 = out_w.transpose(2, 3, 1, 0).reshape(9 * E, E).astype(jnp.bfloat16)
    b1 = conv_b.reshape(1, 5 * E)
    b2 = proj_b.reshape(1, 3 * E)
    b3 = out_b.reshape(1, E)
    posw = pos_w.reshape(MEM, NH, THD)
    lnw = ln_w.reshape(E, S).T
    lnb = ln_b.reshape(E, S).T

    blk = lambda shp: pl.BlockSpec(shp, lambda i: (i,) + (0,) * (len(shp) - 1))
    full = lambda arr: pl.BlockSpec(arr.shape, lambda i: (0,) * arr.ndim)

    h_n, c_n, k_o, v_o = pl.pallas_call(
        _cell_kernel,
        grid=(GRID,),
        in_specs=[
            blk((BB, S, C)), blk((BB, S, E)), blk((BB, S, E)),
            blk((BB, MEM, NH, THD)), blk((BB, MEM, NH, THD)),
            blk((BB, MEM, NH)),
            full(wgk), full(w3),
            full(b1), full(b2), full(b3),
            full(posw), full(pos_b), full(lnw), full(lnb),
        ],
        out_specs=[
            blk((BB, S, E)), blk((BB, S, E)),
            blk((BB, MEM, NH, THD)), blk((BB, MEM, NH, THD)),
        ],
        out_shape=[
            jax.ShapeDtypeStruct((B, S, E), jnp.float32),
            jax.ShapeDtypeStruct((B, S, E), jnp.float32),
            jax.ShapeDtypeStruct((B, MEM, NH, THD), jnp.float32),
            jax.ShapeDtypeStruct((B, MEM, NH, THD), jnp.float32),
        ],
        compiler_params=pltpu.CompilerParams(
            dimension_semantics=("arbitrary",),
            vmem_limit_bytes=56 * 1024 * 1024,
        ),
        name="conv_attn_lstm_cell",
    )(x, h, c, concat_k, concat_v, maskf, wgk, w3, b1, b2, b3,
      posw, pos_b, lnw, lnb)
    return (h_n.transpose(0, 2, 1).reshape(B, E, H, W),
            c_n.transpose(0, 2, 1).reshape(B, E, H, W), k_o, v_o)
